# Initial kernel scaffold; baseline (speedup 1.0000x reference)
import functools

import jax
import jax.numpy as jnp
from jax import lax
from jax.experimental import pallas as pl
from jax.experimental.pallas import tpu as pltpu
from jax.experimental.pallas import tpu_sc as plsc

N = 10000
E = 320000
D = 128

NC = 2
NS = 16
NW = NC * NS
EW = E // NW
C = 80
CH = EW // C
RPS = N // NS
DGW = 8


def _sc_agg_body(with_deg, *refs):
    if with_deg:
        (feat, srcr, dstr, z2d, zdeg, ones_h,
         out_agg, out_deg,
         src_v, dst_v, rows_v, ones_v, sh_agg, sh_deg, sem) = refs
    else:
        (feat, srcr, dstr, z2d,
         out_agg,
         src_v, dst_v, rows_v, sh_agg, sem) = refs

    cid = lax.axis_index("c")
    sid = lax.axis_index("s")
    wid = sid * NC + cid

    pltpu.sync_copy(z2d.at[pl.ds(sid * RPS, RPS)],
                    sh_agg.at[pl.ds(sid * RPS, RPS)])
    if with_deg:
        pltpu.sync_copy(zdeg.at[pl.ds(sid * RPS, RPS)],
                        sh_deg.at[pl.ds(sid * RPS, RPS)])
        pltpu.sync_copy(ones_h, ones_v)

    pltpu.sync_copy(srcr.at[wid], src_v)
    pltpu.sync_copy(dstr.at[wid], dst_v)
    plsc.subcore_barrier()

    def chunk(k, carry):
        pltpu.async_copy(feat.at[src_v.at[k]], rows_v, sem).wait()
        pltpu.sync_copy(rows_v, sh_agg.at[dst_v.at[k]], add=True)
        if with_deg:
            pltpu.sync_copy(ones_v, sh_deg.at[dst_v.at[k]], add=True)
        return carry

    lax.fori_loop(0, CH, chunk, 0)
    plsc.subcore_barrier()

    pltpu.sync_copy(sh_agg.at[pl.ds(sid * RPS, RPS)],
                    out_agg.at[cid, pl.ds(sid * RPS, RPS)])
    if with_deg:
        pltpu.sync_copy(sh_deg.at[pl.ds(sid * RPS, RPS)],
                        out_deg.at[cid, pl.ds(sid * RPS, RPS)])


def _make_sc_agg(with_deg):
    out_type = [jax.ShapeDtypeStruct((NC, N, D), jnp.float32)]
    scratch = [
        pltpu.VMEM((CH, C), jnp.int32),
        pltpu.VMEM((CH, C), jnp.int32),
        pltpu.VMEM((C, D), jnp.float32),
    ]
    if with_deg:
        out_type.append(jax.ShapeDtypeStruct((NC, N, DGW), jnp.float32))
        scratch.append(pltpu.VMEM((C, DGW), jnp.float32))
    scratch.append(pltpu.VMEM_SHARED((N, D), jnp.float32))
    if with_deg:
        scratch.append(pltpu.VMEM_SHARED((N, DGW), jnp.float32))
    scratch.append(pltpu.SemaphoreType.DMA)

    return pl.kernel(
        functools.partial(_sc_agg_body, with_deg),
        out_type=out_type,
        mesh=plsc.VectorSubcoreMesh(core_axis_name="c", subcore_axis_name="s"),
        scratch_types=scratch,
    )


def _tc_layer_body(x_ref, p_ref, d_ref, w_ref, b_ref, o_ref):
    deg = d_ref[0, :, :1] + d_ref[1, :, :1]
    inv = 1.0 / jnp.maximum(deg, 1.0)
    agg = (p_ref[0] + p_ref[1]) * inv
    acc = jnp.dot(x_ref[...], w_ref[:D], preferred_element_type=jnp.float32)
    acc = acc + jnp.dot(agg, w_ref[D:], preferred_element_type=jnp.float32)
    o_ref[...] = jnp.maximum(acc + b_ref[...], 0.0)


def _tc_layer(x, parts, degp, W, b2d, blk=2000):
    grid = (N // blk,)
    return pl.pallas_call(
        _tc_layer_body,
        grid=grid,
        in_specs=[
            pl.BlockSpec((blk, D), lambda i: (i, 0)),
            pl.BlockSpec((NC, blk, D), lambda i: (0, i, 0)),
            pl.BlockSpec((NC, blk, DGW), lambda i: (0, i, 0)),
            pl.BlockSpec((2 * D, D), lambda i: (0, 0)),
            pl.BlockSpec((1, D), lambda i: (0, 0)),
        ],
        out_specs=pl.BlockSpec((blk, D), lambda i: (i, 0)),
        out_shape=jax.ShapeDtypeStruct((N, D), jnp.float32),
    )(x, parts, degp, W, b2d)


_sc_agg_deg = _make_sc_agg(True)
_sc_agg = _make_sc_agg(False)


@jax.jit
def kernel(x, edge_index, W1, b1, W2, b2):
    src = edge_index[0].reshape(NW, CH, C)
    dst = edge_index[1].reshape(NW, CH, C)
    z2d = jnp.zeros((N, D), jnp.float32)
    zdeg = jnp.zeros((N, DGW), jnp.float32)
    ones8 = jnp.ones((C, DGW), jnp.float32)

    parts1, degp = _sc_agg_deg(x, src, dst, z2d, zdeg, ones8)
    h = _tc_layer(x, parts1, degp, W1, b1.reshape(1, D))
    parts2 = _sc_agg(h, src, dst, z2d)
    return _tc_layer(h, parts2, degp, W2, b2.reshape(1, D))


# speedup vs baseline: 6.3100x; 6.3100x over previous
import functools

import jax
import jax.numpy as jnp
from jax import lax
from jax.experimental import pallas as pl
from jax.experimental.pallas import tpu as pltpu
from jax.experimental.pallas import tpu_sc as plsc

N = 10000
E = 320000
D = 128

NC = 2
NS = 16
NW = NC * NS
EW = E // NW
C = 80
G = 5
IB = 25
CH = G * IB
NP = 10240
RPS = NP // NS


def _sc_agg_body(const_rows, *refs):
    if const_rows:
        (ones_cd, dstr, z2d, out_agg, dst_v, rows_v, sh_agg) = refs
    else:
        (feat, srcr, dstr, z2d, out_agg, src_v, dst_v, rows_v, sh_agg) = refs

    cid = lax.axis_index("c")
    sid = lax.axis_index("s")
    wid = sid * NC + cid

    pltpu.sync_copy(z2d, rows_v)

    def zero_blk(j, carry):
        pltpu.sync_copy(rows_v, sh_agg.at[pl.ds(sid * RPS + j * C, C)])
        return carry

    lax.fori_loop(0, RPS // C, zero_blk, 0)
    if const_rows:
        pltpu.sync_copy(ones_cd, rows_v)
    plsc.subcore_barrier()

    def superchunk(g, carry):
        if not const_rows:
            pltpu.sync_copy(srcr.at[wid, g], src_v)
        pltpu.sync_copy(dstr.at[wid, g], dst_v)

        def chunk(k, carry2):
            if not const_rows:
                pltpu.sync_copy(feat.at[src_v.at[k]], rows_v)
            pltpu.sync_copy(rows_v, sh_agg.at[dst_v.at[k]], add=True)
            return carry2

        return lax.fori_loop(0, IB, chunk, carry)

    lax.fori_loop(0, G, superchunk, 0)
    plsc.subcore_barrier()

    def emit_blk(j, carry):
        base = sid * RPS + j * C
        pltpu.sync_copy(sh_agg.at[pl.ds(base, C)], rows_v)
        pltpu.sync_copy(rows_v, out_agg.at[cid, pl.ds(base, C)])
        return carry

    lax.fori_loop(0, RPS // C, emit_blk, 0)


def _make_sc_agg(const_rows):
    scratch = []
    if not const_rows:
        scratch.append(pltpu.VMEM((IB, C), jnp.int32))
    scratch.extend([
        pltpu.VMEM((IB, C), jnp.int32),
        pltpu.VMEM((C, D), jnp.float32),
        pltpu.VMEM_SHARED((NP, D), jnp.float32),
    ])
    return pl.kernel(
        functools.partial(_sc_agg_body, const_rows),
        out_type=[jax.ShapeDtypeStruct((NC, NP, D), jnp.float32)],
        mesh=plsc.VectorSubcoreMesh(core_axis_name="c", subcore_axis_name="s"),
        scratch_types=scratch,
    )


def _tc_layer_body(x_ref, p_ref, d_ref, w_ref, b_ref, o_ref):
    deg = d_ref[0, :, :1] + d_ref[1, :, :1]
    inv = 1.0 / jnp.maximum(deg, 1.0)
    agg = (p_ref[0] + p_ref[1]) * inv
    acc = jnp.dot(x_ref[...], w_ref[:D], preferred_element_type=jnp.float32)
    acc = acc + jnp.dot(agg, w_ref[D:], preferred_element_type=jnp.float32)
    o_ref[...] = jnp.maximum(acc + b_ref[...], 0.0)


def _tc_layer(x, parts, degp, W, b2d, blk=2000):
    grid = (N // blk,)
    return pl.pallas_call(
        _tc_layer_body,
        grid=grid,
        in_specs=[
            pl.BlockSpec((blk, D), lambda i: (i, 0)),
            pl.BlockSpec((NC, blk, D), lambda i: (0, i, 0)),
            pl.BlockSpec((NC, blk, D), lambda i: (0, i, 0)),
            pl.BlockSpec((2 * D, D), lambda i: (0, 0)),
            pl.BlockSpec((1, D), lambda i: (0, 0)),
        ],
        out_specs=pl.BlockSpec((blk, D), lambda i: (i, 0)),
        out_shape=jax.ShapeDtypeStruct((N, D), jnp.float32),
    )(x, parts, degp, W, b2d)


_sc_agg = _make_sc_agg(False)
_sc_deg = _make_sc_agg(True)


@jax.jit
def kernel(x, edge_index, W1, b1, W2, b2):
    src = edge_index[0].reshape(NW, G, IB, C)
    dst = edge_index[1].reshape(NW, G, IB, C)
    z2d = jnp.zeros((C, D), jnp.float32)
    ones_cd = jnp.ones((C, D), jnp.float32)

    (degp,) = _sc_deg(ones_cd, dst, z2d)
    (parts1,) = _sc_agg(x, src, dst, z2d)
    h = _tc_layer(x, parts1, degp, W1, b1.reshape(1, D))
    (parts2,) = _sc_agg(h, src, dst, z2d)
    return _tc_layer(h, parts2, degp, W2, b2.reshape(1, D))
